# baseline (speedup 1.0000x reference)
import functools

import jax
import jax.numpy as jnp
from jax import lax
from jax.experimental import pallas as pl
from jax.experimental.pallas import tpu as pltpu
from jax.experimental.pallas import tpu_sc as plsc

N_REGIONS = 6
BOUND = 2.5

_B, _C, _H, _W = 8, 96, 224, 224
_N = _B * _C * _H * _W
_ROW = _H * _W
_NW = 32
_PER_W = _N // _NW
_CHUNK = _ROW
_CHUNKS_PER_W = _PER_W // _CHUNK
_ROWS_PER_W = _PER_W // _ROW
_NROWS = _B * _C
_TAB = _NROWS * 16


def _sc_body(x_hbm, ct_hbm, out_hbm, in_v, out_v, ct_tab):
    wid = lax.axis_index("s") * 2 + lax.axis_index("c")
    pltpu.sync_copy(ct_hbm, ct_tab)

    def chunk_body(k, _):
        row = wid * _ROWS_PER_W + k
        off = row * _CHUNK
        pltpu.sync_copy(x_hbm.at[pl.ds(off, _CHUNK)], in_v)
        cvec = ct_tab[pl.ds(row * 16, 16)]
        av = [jnp.full((16,), cvec[j]) for j in range(N_REGIONS)]
        bv = [jnp.full((16,), cvec[8 + j]) for j in range(N_REGIONS)]

        @plsc.parallel_loop(0, _CHUNK, step=16, unroll=8)
        def vec_body(i):
            v = in_v[pl.ds(i, 16)]
            s = v * (0.5 * N_REGIONS / BOUND) + (0.5 * N_REGIONS)
            a = av[0]
            b = bv[0]
            for j in range(1, N_REGIONS):
                m = s >= float(j)
                a = jnp.where(m, av[j], a)
                b = jnp.where(m, bv[j], b)
            out_v[pl.ds(i, 16)] = a + b * s

        pltpu.sync_copy(out_v, out_hbm.at[pl.ds(off, _CHUNK)])
        return 0

    lax.fori_loop(0, _CHUNKS_PER_W, chunk_body, 0)


def kernel(x, points):
    B, C, H, W = x.shape

    slopes = points[:, 1:] - points[:, :-1]
    intercepts = points[:, :-1] - slopes * jnp.arange(
        N_REGIONS, dtype=points.dtype
    )[None, :]
    ct = jnp.zeros((C, 16), dtype=points.dtype)
    ct = ct.at[:, :N_REGIONS].set(intercepts)
    ct = ct.at[:, 8:8 + N_REGIONS].set(slopes)
    ct_flat = jnp.tile(ct, (B, 1)).reshape(-1)

    xf = x.reshape(-1)
    sc_kernel = functools.partial(
        pl.kernel,
        out_type=jax.ShapeDtypeStruct((_N,), jnp.float32),
        mesh=plsc.VectorSubcoreMesh(core_axis_name="c", subcore_axis_name="s"),
        compiler_params=pltpu.CompilerParams(needs_layout_passes=False),
        scratch_types=[
            pltpu.VMEM((_CHUNK,), jnp.float32),
            pltpu.VMEM((_CHUNK,), jnp.float32),
            pltpu.VMEM((_TAB,), jnp.float32),
        ],
    )(_sc_body)
    out = sc_kernel(xf, ct_flat)
    return out.reshape(B, C, H, W)


# speedup vs baseline: 1.3964x; 1.3964x over previous
import functools

import jax
import jax.numpy as jnp
from jax import lax
from jax.experimental import pallas as pl
from jax.experimental.pallas import tpu as pltpu
from jax.experimental.pallas import tpu_sc as plsc

N_REGIONS = 6
BOUND = 2.5

_B, _C, _H, _W = 8, 96, 224, 224
_N = _B * _C * _H * _W
_ROW = _H * _W
_NW = 32
_PER_W = _N // _NW
_CHUNK = _ROW // 2
_CHUNKS_PER_W = _PER_W // _CHUNK
_ROWS_PER_W = _PER_W // _ROW
_TAB = _B * _C * N_REGIONS


def _sc_body(x_hbm, a_hbm, b_hbm, out_hbm,
             in_v0, in_v1, out_v, a_tab, b_tab, sem0, sem1):
    wid = lax.axis_index("s") * 2 + lax.axis_index("c")
    pltpu.sync_copy(a_hbm, a_tab)
    pltpu.sync_copy(b_hbm, b_tab)
    w_off = wid * _PER_W
    pltpu.async_copy(x_hbm.at[pl.ds(w_off, _CHUNK)], in_v0, sem0)

    def chunk_body(it, _):
        for b in range(2):
            in_v = in_v0 if b == 0 else in_v1
            sem = sem0 if b == 0 else sem1
            k = it * 2 + b
            off = w_off + k * _CHUNK
            base = (wid * _ROWS_PER_W * 2 + k) // 2 * N_REGIONS
            pltpu.make_async_copy(
                x_hbm.at[pl.ds(off, _CHUNK)], in_v, sem).wait()

            nxt = k + 1

            @pl.when(nxt < _CHUNKS_PER_W)
            def _():
                n_in = in_v1 if b == 0 else in_v0
                n_sem = sem1 if b == 0 else sem0
                pltpu.async_copy(
                    x_hbm.at[pl.ds(w_off + nxt * _CHUNK, _CHUNK)],
                    n_in, n_sem)

            @plsc.parallel_loop(0, _CHUNK, step=16, unroll=8)
            def vec_body(i):
                v = in_v[pl.ds(i, 16)]
                s = v * (0.5 * N_REGIONS / BOUND) + (0.5 * N_REGIONS)
                sc = jnp.minimum(jnp.maximum(s, 0.0),
                                 float(N_REGIONS) * 0.999)
                idx = sc.astype(jnp.int32) + base
                a = plsc.load_gather(a_tab, [idx])
                bb = plsc.load_gather(b_tab, [idx])
                out_v[pl.ds(i, 16)] = a + bb * s

            pltpu.sync_copy(out_v, out_hbm.at[pl.ds(off, _CHUNK)])
        return 0

    lax.fori_loop(0, _CHUNKS_PER_W // 2, chunk_body, 0)


def kernel(x, points):
    B, C, H, W = x.shape

    slopes = points[:, 1:] - points[:, :-1]
    intercepts = points[:, :-1] - slopes * jnp.arange(
        N_REGIONS, dtype=points.dtype
    )[None, :]
    a_flat = jnp.tile(intercepts, (B, 1)).reshape(-1)
    b_flat = jnp.tile(slopes, (B, 1)).reshape(-1)

    xf = x.reshape(-1)
    sc_kernel = functools.partial(
        pl.kernel,
        out_type=jax.ShapeDtypeStruct((_N,), jnp.float32),
        mesh=plsc.VectorSubcoreMesh(core_axis_name="c", subcore_axis_name="s"),
        compiler_params=pltpu.CompilerParams(needs_layout_passes=False),
        scratch_types=[
            pltpu.VMEM((_CHUNK,), jnp.float32),
            pltpu.VMEM((_CHUNK,), jnp.float32),
            pltpu.VMEM((_CHUNK,), jnp.float32),
            pltpu.VMEM((_TAB,), jnp.float32),
            pltpu.VMEM((_TAB,), jnp.float32),
            pltpu.SemaphoreType.DMA,
            pltpu.SemaphoreType.DMA,
        ],
    )(_sc_body)
    out = sc_kernel(xf, a_flat, b_flat)
    return out.reshape(B, C, H, W)


# speedup vs baseline: 1.5059x; 1.0784x over previous
import functools

import jax
import jax.numpy as jnp
from jax import lax
from jax.experimental import pallas as pl
from jax.experimental.pallas import tpu as pltpu
from jax.experimental.pallas import tpu_sc as plsc

N_REGIONS = 6
BOUND = 2.5

_B, _C, _H, _W = 8, 96, 224, 224
_N = _B * _C * _H * _W
_ROW = _H * _W
_NW = 32
_PER_W = _N // _NW
_CHUNK = _ROW // 2
_CHUNKS_PER_W = _PER_W // _CHUNK
_ROWS_PER_W = _PER_W // _ROW
_TAB = _B * _C * N_REGIONS


def _sc_body(x_hbm, a_hbm, b_hbm, out_hbm,
             in_v0, in_v1, out_v0, out_v1, a_tab, b_tab,
             sem0, sem1, osem0, osem1):
    wid = lax.axis_index("s") * 2 + lax.axis_index("c")
    pltpu.sync_copy(a_hbm, a_tab)
    pltpu.sync_copy(b_hbm, b_tab)
    w_off = wid * _PER_W
    pltpu.async_copy(x_hbm.at[pl.ds(w_off, _CHUNK)], in_v0, sem0)

    def chunk_body(it, _):
        for b in range(2):
            in_v = in_v0 if b == 0 else in_v1
            sem = sem0 if b == 0 else sem1
            out_v = out_v0 if b == 0 else out_v1
            osem = osem0 if b == 0 else osem1
            k = it * 2 + b
            off = w_off + k * _CHUNK
            base = (wid * _ROWS_PER_W * 2 + k) // 2 * N_REGIONS
            pltpu.make_async_copy(
                x_hbm.at[pl.ds(off, _CHUNK)], in_v, sem).wait()

            nxt = k + 1

            @pl.when(nxt < _CHUNKS_PER_W)
            def _():
                n_in = in_v1 if b == 0 else in_v0
                n_sem = sem1 if b == 0 else sem0
                pltpu.async_copy(
                    x_hbm.at[pl.ds(w_off + nxt * _CHUNK, _CHUNK)],
                    n_in, n_sem)

            @pl.when(k >= 2)
            def _():
                pltpu.make_async_copy(
                    out_v, out_hbm.at[pl.ds(off - 2 * _CHUNK, _CHUNK)],
                    osem).wait()

            @plsc.parallel_loop(0, _CHUNK, step=16, unroll=8)
            def vec_body(i):
                v = in_v[pl.ds(i, 16)]
                s = v * (0.5 * N_REGIONS / BOUND) + (0.5 * N_REGIONS)
                sc = jnp.minimum(jnp.maximum(s, 0.0),
                                 float(N_REGIONS) * 0.999)
                idx = sc.astype(jnp.int32) + base
                a = plsc.load_gather(a_tab, [idx])
                bb = plsc.load_gather(b_tab, [idx])
                out_v[pl.ds(i, 16)] = a + bb * s

            pltpu.async_copy(out_v, out_hbm.at[pl.ds(off, _CHUNK)], osem)
        return 0

    lax.fori_loop(0, _CHUNKS_PER_W // 2, chunk_body, 0)
    last0 = w_off + (_CHUNKS_PER_W - 2) * _CHUNK
    last1 = w_off + (_CHUNKS_PER_W - 1) * _CHUNK
    pltpu.make_async_copy(
        out_v0, out_hbm.at[pl.ds(last0, _CHUNK)], osem0).wait()
    pltpu.make_async_copy(
        out_v1, out_hbm.at[pl.ds(last1, _CHUNK)], osem1).wait()


def kernel(x, points):
    B, C, H, W = x.shape

    slopes = points[:, 1:] - points[:, :-1]
    intercepts = points[:, :-1] - slopes * jnp.arange(
        N_REGIONS, dtype=points.dtype
    )[None, :]
    a_flat = jnp.tile(intercepts, (B, 1)).reshape(-1)
    b_flat = jnp.tile(slopes, (B, 1)).reshape(-1)

    xf = x.reshape(-1)
    sc_kernel = functools.partial(
        pl.kernel,
        out_type=jax.ShapeDtypeStruct((_N,), jnp.float32),
        mesh=plsc.VectorSubcoreMesh(core_axis_name="c", subcore_axis_name="s"),
        compiler_params=pltpu.CompilerParams(needs_layout_passes=False),
        scratch_types=[
            pltpu.VMEM((_CHUNK,), jnp.float32),
            pltpu.VMEM((_CHUNK,), jnp.float32),
            pltpu.VMEM((_CHUNK,), jnp.float32),
            pltpu.VMEM((_CHUNK,), jnp.float32),
            pltpu.VMEM((_TAB,), jnp.float32),
            pltpu.VMEM((_TAB,), jnp.float32),
            pltpu.SemaphoreType.DMA,
            pltpu.SemaphoreType.DMA,
            pltpu.SemaphoreType.DMA,
            pltpu.SemaphoreType.DMA,
        ],
    )(_sc_body)
    out = sc_kernel(xf, a_flat, b_flat)
    return out.reshape(B, C, H, W)
